# baseline (speedup 1.0000x reference)
import functools

import jax
import jax.numpy as jnp
from jax.experimental import pallas as pl


def _body(cls_ref, reg_ref, anch_ref, ann_ref, annT_ref, out_ref, *, N, C, M, TN):
    nb = pl.program_id(1)

    a = anch_ref[...]
    ax, ay, aal = a[0:1, :], a[1:2, :], a[2:3, :]
    g = ann_ref[0]
    gx, gy, gal = g[:, 0:1], g[:, 1:2], g[:, 2:3]

    dx = gx - ax
    dy = gy - ay
    dxy = jnp.sqrt(dx * dx + dy * dy)
    d = 10.0 * dxy + jnp.abs(gal - aal)

    dmin = jnp.min(d, axis=0, keepdims=True)
    iota_m = jax.lax.broadcasted_iota(jnp.int32, (M, TN), 0)
    ids = jnp.where(d == dmin, iota_m, M)
    amin = jnp.min(ids, axis=0, keepdims=True)
    onehot = (iota_m == amin).astype(jnp.float32)
    fields = jax.lax.dot_general(
        annT_ref[0], onehot, (((1,), (0,)), ((), ())),
        preferred_element_type=jnp.float32)
    bx, by, bcls = fields[0:1], fields[1:2], fields[2:3]
    bcos, bsin = fields[3:4], fields[4:5]

    pos = dmin <= 110.0
    neg = dmin >= 130.0
    lane = jax.lax.broadcasted_iota(jnp.int32, (1, TN), 1) + nb * TN
    valid = lane < N
    posv = jnp.logical_and(pos, valid)
    negv = jnp.logical_and(neg, valid)

    p = jnp.clip(jnp.transpose(cls_ref[0], (1, 0)), 0.0001, 1.0 - 0.0001)
    w_neg = 0.75 * p * p * (-jnp.log(1.0 - p))
    S = jnp.sum(w_neg, axis=0, keepdims=True)
    iota_c = jax.lax.broadcasted_iota(jnp.int32, (C, TN), 0)
    bcls_i = bcls.astype(jnp.int32)
    p_sel = jnp.sum(jnp.where(iota_c == bcls_i, p, 0.0), axis=0, keepdims=True)
    w_neg_sel = 0.75 * p_sel * p_sel * (-jnp.log(1.0 - p_sel))
    w_pos_sel = 0.25 * (1.0 - p_sel) * (1.0 - p_sel) * (-jnp.log(p_sel))
    cls_l = jnp.where(posv, S - w_neg_sel + w_pos_sel,
                      jnp.where(negv, S, 0.0))
    np_l = jnp.where(posv, 1.0, 0.0)

    r = reg_ref[0]
    rdx = jnp.abs((bx - ax) - r[0:1])
    rdy = jnp.abs((by - ay) - r[1:2])
    lx = jnp.where(rdx <= 1.0 / 9.0, 0.5 * 9.0 * rdx * rdx, rdx - 0.5 / 9.0)
    ly = jnp.where(rdy <= 1.0 / 9.0, 0.5 * 9.0 * rdy * rdy, rdy - 0.5 / 9.0)
    ang = 1.0 - (bcos * r[2:3] + bsin * r[3:4])
    xy_l = jnp.where(posv, lx + ly, 0.0)
    ang_l = jnp.where(posv, ang, 0.0)

    def red128(v):
        acc = v[:, 0:128]
        for k in range(1, TN // 128):
            acc = acc + v[:, k * 128:(k + 1) * 128]
        return acc

    tot = jnp.concatenate(
        [red128(cls_l), red128(np_l), red128(xy_l), red128(ang_l)], axis=0)

    @pl.when(nb == 0)
    def _():
        out_ref[0] = tot

    @pl.when(nb != 0)
    def _():
        out_ref[0] = out_ref[0] + tot


def kernel(classifications, regressions, anchors, annotations):
    B, N, C = classifications.shape
    M = annotations.shape[1]
    TN = 1024
    NB = pl.cdiv(N, TN)

    anchT = jnp.transpose(anchors[0], (1, 0))
    phi = anchors[0, :, 2][None, :] + regressions[:, :, 2]
    regT = jnp.concatenate(
        [jnp.transpose(regressions[:, :, :2], (0, 2, 1)),
         jnp.cos(phi)[:, None, :], jnp.sin(phi)[:, None, :]], axis=1)
    annT0 = jnp.transpose(annotations, (0, 2, 1))
    gal = annotations[:, :, 2]
    annT = jnp.concatenate(
        [annT0[:, 0:2, :], annT0[:, 3:4, :],
         jnp.cos(gal)[:, None, :], jnp.sin(gal)[:, None, :]], axis=1)

    out = pl.pallas_call(
        functools.partial(_body, N=N, C=C, M=M, TN=TN),
        grid=(B, NB),
        in_specs=[
            pl.BlockSpec((1, TN, C), lambda b, n: (b, n, 0)),
            pl.BlockSpec((1, 4, TN), lambda b, n: (b, 0, n)),
            pl.BlockSpec((3, TN), lambda b, n: (0, n)),
            pl.BlockSpec((1, M, 4), lambda b, n: (b, 0, 0)),
            pl.BlockSpec((1, 5, M), lambda b, n: (b, 0, 0)),
        ],
        out_specs=pl.BlockSpec((1, 4, 128), lambda b, n: (b, 0, 0)),
        out_shape=jax.ShapeDtypeStruct((B, 4, 128), jnp.float32),
    )(classifications, regT, anchT, annotations, annT)

    s = out.sum(axis=-1)
    cls_sum, npos, sxy, sang = s[:, 0], s[:, 1], s[:, 2], s[:, 3]
    cnt = jnp.maximum(npos, 1.0)
    cls_loss = cls_sum / cnt
    reg_loss = jnp.where(npos > 0, sxy / (2.0 * cnt) + sang / cnt, 0.0)
    return (jnp.mean(cls_loss, keepdims=True), jnp.mean(reg_loss, keepdims=True))


# speedup vs baseline: 1.2606x; 1.2606x over previous
import functools

import jax
import jax.numpy as jnp
from jax.experimental import pallas as pl


def _body(cls_ref, reg_ref, anch_ref, anchC_ref, ann_ref, annT_ref, out_ref,
          *, N, C, M, TN):
    nb = pl.program_id(1)
    TC8 = TN // 8

    a = anch_ref[...]
    ax, ay, aal = a[0:1, :], a[1:2, :], a[2:3, :]
    g = ann_ref[0]
    gx, gy, gal = g[:, 0:1], g[:, 1:2], g[:, 2:3]

    dx = gx - ax
    dy = gy - ay
    dxy = jnp.sqrt(dx * dx + dy * dy)
    d = 10.0 * dxy + jnp.abs(gal - aal)

    dmin = jnp.min(d, axis=0, keepdims=True)
    iota_m = jax.lax.broadcasted_iota(jnp.int32, (M, 1), 0).astype(jnp.float32)
    ids = jnp.where(d == dmin, iota_m, float(M))
    amin = jnp.min(ids, axis=0, keepdims=True)
    onehot = jnp.where(ids == amin, 1.0, 0.0)
    fields = jax.lax.dot_general(
        annT_ref[0], onehot, (((1,), (0,)), ((), ())),
        preferred_element_type=jnp.float32)

    dmin8 = jnp.reshape(dmin, (8, TC8))
    bx = jnp.reshape(fields[0:1], (8, TC8))
    by = jnp.reshape(fields[1:2], (8, TC8))
    bcos = jnp.reshape(fields[3:4], (8, TC8))
    bsin = jnp.reshape(fields[4:5], (8, TC8))

    pos = dmin8 <= 110.0
    neg = dmin8 >= 130.0
    lane = jax.lax.broadcasted_iota(jnp.int32, (8, TC8), 0) * TC8 \
        + jax.lax.broadcasted_iota(jnp.int32, (8, TC8), 1) + nb * TN
    valid = lane < N
    posv = jnp.logical_and(pos, valid)
    negv = jnp.logical_and(neg, valid)

    p = jnp.clip(cls_ref[0], 0.0001, 1.0 - 0.0001)
    w_neg = 0.75 * p * p * (-jnp.log(1.0 - p))
    S = jnp.reshape(jnp.sum(w_neg, axis=0, keepdims=True), (8, TC8))
    iota_c = jax.lax.broadcasted_iota(jnp.int32, (C, 1), 0)
    bcls_i = fields[2:3].astype(jnp.int32)
    p_sel = jnp.reshape(
        jnp.sum(jnp.where(iota_c == bcls_i, p, 0.0), axis=0, keepdims=True),
        (8, TC8))
    w_neg_sel = 0.75 * p_sel * p_sel * (-jnp.log(1.0 - p_sel))
    w_pos_sel = 0.25 * (1.0 - p_sel) * (1.0 - p_sel) * (-jnp.log(p_sel))
    cls_l = jnp.where(posv, S - w_neg_sel + w_pos_sel,
                      jnp.where(negv, S, 0.0))
    np_l = jnp.where(posv, 1.0, 0.0)

    r = reg_ref[0, :, 0]
    ac = anchC_ref[:, 0]
    rdx = jnp.abs((bx - ac[0]) - r[0])
    rdy = jnp.abs((by - ac[1]) - r[1])
    lx = jnp.where(rdx <= 1.0 / 9.0, 0.5 * 9.0 * rdx * rdx, rdx - 0.5 / 9.0)
    ly = jnp.where(rdy <= 1.0 / 9.0, 0.5 * 9.0 * rdy * rdy, rdy - 0.5 / 9.0)
    ang = 1.0 - (bcos * r[2] + bsin * r[3])
    xy_l = jnp.where(posv, lx + ly, 0.0)
    ang_l = jnp.where(posv, ang, 0.0)

    def red(v):
        acc = v[:, 0:128]
        for k in range(1, TC8 // 128):
            acc = acc + v[:, k * 128:(k + 1) * 128]
        return acc

    tot = jnp.concatenate(
        [red(cls_l), red(np_l), red(xy_l), red(ang_l)], axis=0)

    @pl.when(nb == 0)
    def _():
        out_ref[0] = tot

    @pl.when(nb != 0)
    def _():
        out_ref[0] = out_ref[0] + tot


def kernel(classifications, regressions, anchors, annotations):
    B, N, C = classifications.shape
    M = annotations.shape[1]
    TN = 1024
    NB = pl.cdiv(N, TN)
    Npad = NB * TN

    clsT = jnp.transpose(classifications, (0, 2, 1))
    anchT = jnp.transpose(anchors[0], (1, 0))
    anchC = jnp.reshape(
        jnp.pad(anchT, ((0, 0), (0, Npad - N))), (3, NB, 8, TN // 8))
    phi = anchors[0, :, 2][None, :] + regressions[:, :, 2]
    regT = jnp.concatenate(
        [jnp.transpose(regressions[:, :, :2], (0, 2, 1)),
         jnp.cos(phi)[:, None, :], jnp.sin(phi)[:, None, :]], axis=1)
    regC = jnp.reshape(
        jnp.pad(regT, ((0, 0), (0, 0), (0, Npad - N))), (B, 4, NB, 8, TN // 8))
    annT0 = jnp.transpose(annotations, (0, 2, 1))
    gal = annotations[:, :, 2]
    annT = jnp.concatenate(
        [annT0[:, 0:2, :], annT0[:, 3:4, :],
         jnp.cos(gal)[:, None, :], jnp.sin(gal)[:, None, :]], axis=1)

    out = pl.pallas_call(
        functools.partial(_body, N=N, C=C, M=M, TN=TN),
        grid=(B, NB),
        in_specs=[
            pl.BlockSpec((1, C, TN), lambda b, n: (b, 0, n)),
            pl.BlockSpec((1, 4, 1, 8, TN // 8), lambda b, n: (b, 0, n, 0, 0)),
            pl.BlockSpec((3, TN), lambda b, n: (0, n)),
            pl.BlockSpec((3, 1, 8, TN // 8), lambda b, n: (0, n, 0, 0)),
            pl.BlockSpec((1, M, 4), lambda b, n: (b, 0, 0)),
            pl.BlockSpec((1, 5, M), lambda b, n: (b, 0, 0)),
        ],
        out_specs=pl.BlockSpec((1, 32, 128), lambda b, n: (b, 0, 0)),
        out_shape=jax.ShapeDtypeStruct((B, 32, 128), jnp.float32),
    )(clsT, regC, anchT, anchC, annotations, annT)

    s = out.reshape(B, 4, 8 * 128).sum(axis=-1)
    cls_sum, npos, sxy, sang = s[:, 0], s[:, 1], s[:, 2], s[:, 3]
    cnt = jnp.maximum(npos, 1.0)
    cls_loss = cls_sum / cnt
    reg_loss = jnp.where(npos > 0, sxy / (2.0 * cnt) + sang / cnt, 0.0)
    return (jnp.mean(cls_loss, keepdims=True), jnp.mean(reg_loss, keepdims=True))


# speedup vs baseline: 1.5360x; 1.2185x over previous
import functools

import jax
import jax.numpy as jnp
from jax.experimental import pallas as pl


def _body(cls_ref, reg_ref, anch_ref, anchC_ref, ann_ref, annT_ref, out_ref,
          *, N, C, M, TN):
    nb = pl.program_id(1)
    TC8 = TN // 8

    a = anch_ref[...]
    ax, ay, aal = a[0:1, :], a[1:2, :], a[2:3, :]
    g = ann_ref[0]
    gx, gy, gal = g[:, 0:1], g[:, 1:2], g[:, 2:3]

    dx = gx - ax
    dy = gy - ay
    dxy = jnp.sqrt(dx * dx + dy * dy)
    d = 10.0 * dxy + jnp.abs(gal - aal)

    dmin = jnp.min(d, axis=0, keepdims=True)
    iota_m = jax.lax.broadcasted_iota(jnp.int32, (M, 1), 0).astype(jnp.float32)
    ids = jnp.where(d == dmin, iota_m, float(M))
    amin = jnp.min(ids, axis=0, keepdims=True)
    onehot = jnp.where(ids == amin, 1.0, 0.0)
    fields = jax.lax.dot_general(
        annT_ref[0], onehot, (((1,), (0,)), ((), ())),
        preferred_element_type=jnp.float32)

    dmin8 = jnp.reshape(dmin, (8, TC8))
    bx = jnp.reshape(fields[0:1], (8, TC8))
    by = jnp.reshape(fields[1:2], (8, TC8))
    bcos = jnp.reshape(fields[3:4], (8, TC8))
    bsin = jnp.reshape(fields[4:5], (8, TC8))

    pos = dmin8 <= 110.0
    neg = dmin8 >= 130.0
    lane = jax.lax.broadcasted_iota(jnp.int32, (8, TC8), 0) * TC8 \
        + jax.lax.broadcasted_iota(jnp.int32, (8, TC8), 1) + nb * TN
    valid = lane < N
    posv = jnp.logical_and(pos, valid)
    negv = jnp.logical_and(neg, valid)

    p = jnp.clip(cls_ref[0], 0.0001, 1.0 - 0.0001)
    w_neg = 0.75 * p * p * (-jnp.log(1.0 - p))
    S = jnp.reshape(jnp.sum(w_neg, axis=0, keepdims=True), (8, TC8))
    iota_c = jax.lax.broadcasted_iota(jnp.int32, (C, 1), 0)
    bcls_i = fields[2:3].astype(jnp.int32)
    p_sel = jnp.reshape(
        jnp.sum(jnp.where(iota_c == bcls_i, p, 0.0), axis=0, keepdims=True),
        (8, TC8))
    w_neg_sel = 0.75 * p_sel * p_sel * (-jnp.log(1.0 - p_sel))
    w_pos_sel = 0.25 * (1.0 - p_sel) * (1.0 - p_sel) * (-jnp.log(p_sel))
    cls_l = jnp.where(posv, S - w_neg_sel + w_pos_sel,
                      jnp.where(negv, S, 0.0))
    np_l = jnp.where(posv, 1.0, 0.0)

    r = reg_ref[0, :, 0]
    ac = anchC_ref[:, 0]
    rdx = jnp.abs((bx - ac[0]) - r[0])
    rdy = jnp.abs((by - ac[1]) - r[1])
    lx = jnp.where(rdx <= 1.0 / 9.0, 0.5 * 9.0 * rdx * rdx, rdx - 0.5 / 9.0)
    ly = jnp.where(rdy <= 1.0 / 9.0, 0.5 * 9.0 * rdy * rdy, rdy - 0.5 / 9.0)
    ang = 1.0 - (bcos * r[2] + bsin * r[3])
    xy_l = jnp.where(posv, lx + ly, 0.0)
    ang_l = jnp.where(posv, ang, 0.0)

    def red(v):
        acc = v[:, 0:128]
        for k in range(1, TC8 // 128):
            acc = acc + v[:, k * 128:(k + 1) * 128]
        return acc

    tot = jnp.concatenate(
        [red(cls_l), red(np_l), red(xy_l), red(ang_l)], axis=0)

    @pl.when(nb == 0)
    def _():
        out_ref[0] = tot

    @pl.when(nb != 0)
    def _():
        out_ref[0] = out_ref[0] + tot


def kernel(classifications, regressions, anchors, annotations):
    B, N, C = classifications.shape
    M = annotations.shape[1]
    TN = 4096
    NB = pl.cdiv(N, TN)
    Npad = NB * TN

    clsT = jnp.transpose(classifications, (0, 2, 1))
    anchT = jnp.transpose(anchors[0], (1, 0))
    anchC = jnp.reshape(
        jnp.pad(anchT, ((0, 0), (0, Npad - N))), (3, NB, 8, TN // 8))
    phi = anchors[0, :, 2][None, :] + regressions[:, :, 2]
    regT = jnp.concatenate(
        [jnp.transpose(regressions[:, :, :2], (0, 2, 1)),
         jnp.cos(phi)[:, None, :], jnp.sin(phi)[:, None, :]], axis=1)
    regC = jnp.reshape(
        jnp.pad(regT, ((0, 0), (0, 0), (0, Npad - N))), (B, 4, NB, 8, TN // 8))
    annT0 = jnp.transpose(annotations, (0, 2, 1))
    gal = annotations[:, :, 2]
    annT = jnp.concatenate(
        [annT0[:, 0:2, :], annT0[:, 3:4, :],
         jnp.cos(gal)[:, None, :], jnp.sin(gal)[:, None, :]], axis=1)

    out = pl.pallas_call(
        functools.partial(_body, N=N, C=C, M=M, TN=TN),
        grid=(B, NB),
        in_specs=[
            pl.BlockSpec((1, C, TN), lambda b, n: (b, 0, n)),
            pl.BlockSpec((1, 4, 1, 8, TN // 8), lambda b, n: (b, 0, n, 0, 0)),
            pl.BlockSpec((3, TN), lambda b, n: (0, n)),
            pl.BlockSpec((3, 1, 8, TN // 8), lambda b, n: (0, n, 0, 0)),
            pl.BlockSpec((1, M, 4), lambda b, n: (b, 0, 0)),
            pl.BlockSpec((1, 5, M), lambda b, n: (b, 0, 0)),
        ],
        out_specs=pl.BlockSpec((1, 32, 128), lambda b, n: (b, 0, 0)),
        out_shape=jax.ShapeDtypeStruct((B, 32, 128), jnp.float32),
    )(clsT, regC, anchT, anchC, annotations, annT)

    s = out.reshape(B, 4, 8 * 128).sum(axis=-1)
    cls_sum, npos, sxy, sang = s[:, 0], s[:, 1], s[:, 2], s[:, 3]
    cnt = jnp.maximum(npos, 1.0)
    cls_loss = cls_sum / cnt
    reg_loss = jnp.where(npos > 0, sxy / (2.0 * cnt) + sang / cnt, 0.0)
    return (jnp.mean(cls_loss, keepdims=True), jnp.mean(reg_loss, keepdims=True))
